# baseline (speedup 1.0000x reference)
import functools

import jax
import jax.numpy as jnp
from jax import lax
from jax.experimental import pallas as pl
from jax.experimental.pallas import tpu as pltpu
from jax.experimental.pallas import tpu_sc as plsc

_NC = 2
_NS = 16
_NW = _NC * _NS

_CHUNK = 1600
_NSTREAM = 4


@functools.lru_cache(maxsize=None)
def _emb_call(n_total: int, d: int):
    per_w = n_total // _NW
    n_chunks = per_w // _CHUNK
    assert per_w % _CHUNK == 0 and n_total % _NW == 0

    mesh = plsc.VectorSubcoreMesh(core_axis_name="c", subcore_axis_name="s")

    @functools.partial(
        pl.kernel,
        mesh=mesh,
        out_type=jax.ShapeDtypeStruct((n_total, d), jnp.float32),
        compiler_params=pltpu.CompilerParams(use_tc_tiling_on_sc=False),
        scratch_types=[
            pltpu.VMEM((per_w,), jnp.int32),
            pltpu.VMEM((_CHUNK, d), jnp.float32),
            pltpu.VMEM((_CHUNK, d), jnp.float32),
            pltpu.SemaphoreType.DMA,
            pltpu.SemaphoreType.DMA,
            pltpu.SemaphoreType.DMA,
            pltpu.SemaphoreType.DMA,
        ],
    )
    def k(x_hbm, table_hbm, out_hbm, idx_all, rows0, rows1, g0, g1, s0, s1):
        wid = lax.axis_index("s") * _NC + lax.axis_index("c")
        base = wid * per_w
        pltpu.sync_copy(x_hbm.at[pl.ds(base, per_w)], idx_all)

        rows = (rows0, rows1)
        gsem = (g0, g1)
        ssem = (s0, s1)

        def gather_start(g):
            b = g % 2
            sub = _CHUNK // _NSTREAM
            return [
                pltpu.async_copy(
                    table_hbm.at[idx_all.at[pl.ds(g * _CHUNK + j * sub, sub)]],
                    rows[b].at[pl.ds(j * sub, sub)], gsem[b])
                for j in range(_NSTREAM)
            ]

        def store_start(g):
            b = g % 2
            return pltpu.async_copy(
                rows[b], out_hbm.at[pl.ds(base + g * _CHUNK, _CHUNK)],
                ssem[b])

        gathers = [gather_start(0)]
        stores = [None, None]
        for g in range(n_chunks):
            b = g % 2
            if g + 1 < n_chunks:
                if stores[1 - b] is not None:
                    stores[1 - b].wait()
                gathers.append(gather_start(g + 1))
            for h in gathers[g]:
                h.wait()
            stores[b] = store_start(g)
        stores[(n_chunks - 1) % 2].wait()
        if n_chunks > 1:
            stores[n_chunks % 2].wait()

    return k


def kernel(x, table):
    b, s = x.shape
    d = table.shape[1]
    xf = x.reshape(b * s).astype(jnp.int32)
    out = _emb_call(b * s, d)(xf, table)
    return out.reshape(b, s, d)


# speedup vs baseline: 1.6088x; 1.6088x over previous
import functools

import jax
import jax.numpy as jnp
from jax import lax
from jax.experimental import pallas as pl
from jax.experimental.pallas import tpu as pltpu
from jax.experimental.pallas import tpu_sc as plsc

_NC = 2
_NS = 16
_NW = _NC * _NS

_BCHUNK = 32
_NBUF = 2


@functools.lru_cache(maxsize=None)
def _emb_call(b: int, s: int, d: int):
    per_w_b = b // _NW
    per_w = per_w_b * s
    chunk = _BCHUNK * s
    n_chunks = per_w_b // _BCHUNK
    assert per_w_b % _BCHUNK == 0 and n_chunks % _NBUF == 0

    mesh = plsc.VectorSubcoreMesh(core_axis_name="c", subcore_axis_name="s")

    @functools.partial(
        pl.kernel,
        mesh=mesh,
        out_type=jax.ShapeDtypeStruct((b, s, d), jnp.float32),
        compiler_params=pltpu.CompilerParams(use_tc_tiling_on_sc=False),
        scratch_types=[
            pltpu.VMEM((per_w,), jnp.int32),
            pltpu.VMEM((chunk, d), jnp.float32),
            pltpu.VMEM((chunk, d), jnp.float32),
            pltpu.SemaphoreType.DMA,
            pltpu.SemaphoreType.DMA,
            pltpu.SemaphoreType.DMA,
            pltpu.SemaphoreType.DMA,
        ],
    )
    def k(x_hbm, table_hbm, out_hbm, idx_all, rows0, rows1, g0, g1, s0, s1):
        wid = lax.axis_index("s") * _NC + lax.axis_index("c")
        base = wid * per_w
        brow0 = wid * per_w_b
        pltpu.sync_copy(x_hbm.at[pl.ds(base, per_w)], idx_all)

        rows = (rows0, rows1)
        gsem = (g0, g1)
        ssem = (s0, s1)

        def run_chunk(g, bslot):
            @pl.when(g >= _NBUF)
            def _():
                drain = pltpu.make_async_copy(
                    rows[bslot].at[pl.ds(0, s), :], out_hbm.at[0], ssem[bslot])
                for _ in range(_BCHUNK):
                    drain.wait()

            pltpu.async_copy(
                table_hbm.at[idx_all.at[pl.ds(g * chunk, chunk)]],
                rows[bslot], gsem[bslot]).wait()

            for j in range(_BCHUNK):
                pltpu.async_copy(
                    rows[bslot].at[pl.ds(j * s, s), :],
                    out_hbm.at[brow0 + g * _BCHUNK + j], ssem[bslot])

        def body(t, carry):
            for bslot in range(_NBUF):
                run_chunk(t * _NBUF + bslot, bslot)
            return carry

        lax.fori_loop(0, n_chunks // _NBUF, body, 0)

        for bslot in range(_NBUF):
            drain = pltpu.make_async_copy(
                rows[bslot].at[pl.ds(0, s), :], out_hbm.at[0], ssem[bslot])
            for _ in range(_BCHUNK):
                drain.wait()

    return k


def kernel(x, table):
    b, s = x.shape
    d = table.shape[1]
    xf = x.reshape(b * s)
    return _emb_call(b, s, d)(xf, table)


# speedup vs baseline: 1.6103x; 1.0009x over previous
import functools

import jax
import jax.numpy as jnp
from jax import lax
from jax.experimental import pallas as pl
from jax.experimental.pallas import tpu as pltpu
from jax.experimental.pallas import tpu_sc as plsc

_NC = 2
_NS = 16
_NW = _NC * _NS

_BCHUNK = 32
_NBUF = 2


@functools.lru_cache(maxsize=None)
def _emb_call(b: int, s: int, d: int):
    per_w_b = b // _NW
    per_w = per_w_b * s
    chunk = _BCHUNK * s
    n_chunks = per_w_b // _BCHUNK
    assert per_w_b % _BCHUNK == 0 and n_chunks % _NBUF == 0

    mesh = plsc.VectorSubcoreMesh(core_axis_name="c", subcore_axis_name="s")

    @functools.partial(
        pl.kernel,
        mesh=mesh,
        out_type=jax.ShapeDtypeStruct((b, s, d), jnp.float32),
        compiler_params=pltpu.CompilerParams(use_tc_tiling_on_sc=False),
        scratch_types=[
            pltpu.VMEM((per_w,), jnp.int32),
            pltpu.VMEM((chunk, d), jnp.float32),
            pltpu.VMEM((chunk, d), jnp.float32),
            pltpu.SemaphoreType.DMA,
            pltpu.SemaphoreType.DMA,
            pltpu.SemaphoreType.DMA,
            pltpu.SemaphoreType.DMA,
        ],
    )
    def k(x_hbm, table_hbm, out_hbm, idx_all, rows0, rows1, g0, g1, s0, s1):
        wid = lax.axis_index("s") * _NC + lax.axis_index("c")
        base = wid * per_w
        brow0 = wid * per_w_b
        pltpu.sync_copy(x_hbm.at[pl.ds(base, per_w)], idx_all)

        rows = (rows0, rows1)
        gsem = (g0, g1)
        ssem = (s0, s1)

        def run_chunk(g, bslot):
            @pl.when(g >= _NBUF)
            def _():
                drain = pltpu.make_async_copy(
                    rows[bslot].at[pl.ds(0, s), :], out_hbm.at[0], ssem[bslot])
                for _ in range(_BCHUNK):
                    drain.wait()

            pltpu.async_copy(
                table_hbm.at[idx_all.at[pl.ds(g * chunk, chunk)]],
                rows[bslot], gsem[bslot]).wait()

            for j in range(_BCHUNK):
                pltpu.async_copy(
                    rows[bslot].at[pl.ds(j * s, s), :],
                    out_hbm.at[brow0 + g * _BCHUNK + j], ssem[bslot])

        def body(t, carry):
            for bslot in range(_NBUF):
                run_chunk(t * _NBUF + bslot, bslot)
            return carry

        lax.fori_loop(0, n_chunks // _NBUF, body, 0)

        for bslot in range(_NBUF):
            drain = pltpu.make_async_copy(
                rows[bslot].at[pl.ds(0, s), :], out_hbm.at[0], ssem[bslot])
            for _ in range(_BCHUNK):
                drain.wait()

    return k


def kernel(x, table):
    b, s = x.shape
    v, d = table.shape
    xf = jnp.minimum(x.reshape(b * s), v - 1)
    out = _emb_call(b, s, d)(xf, table)
    return out + jnp.float32(0)
